# baseline (speedup 1.0000x reference)
import functools

import jax
import jax.numpy as jnp
from jax import lax
from jax.experimental import pallas as pl
from jax.experimental.pallas import tpu as pltpu
from jax.experimental.pallas import tpu_sc as plsc

N = 10000
D = 128
E = 320000
MIN_NORM = 1e-15
EPS = 4e-3

NC, NS = 2, 16
CH = 128
CPT = 160
E_PAD = NS * CPT * CH
OWN = 5120
DUMP = 512
NA = OWN + DUMP
RZT = NA // NS
RWT = OWN // NS
NB = 10112


def _logmap_body(x_ref, o_ref):
    p = x_ref[...]
    pn = jnp.maximum(jnp.sqrt(jnp.sum(p * p, axis=1, keepdims=True)), MIN_NORM)
    t = jnp.clip(pn, -1.0 + 1e-7, 1.0 - 1e-7)
    o_ref[...] = ((0.5 * jnp.log((1.0 + t) / (1.0 - t))) / pn) * p


def _final_body(p_ref, c_ref, o_ref):
    m = p_ref[...] / jnp.maximum(c_ref[...], 1.0)
    un = jnp.maximum(jnp.sqrt(jnp.sum(m * m, axis=1, keepdims=True)), MIN_NORM)
    e = jnp.tanh(un) * m / un
    en = jnp.maximum(jnp.sqrt(jnp.sum(e * e, axis=1, keepdims=True)), MIN_NORM)
    maxnorm = 1.0 - EPS
    o_ref[...] = jnp.where(en > maxnorm, e / en * maxnorm, e)


def _make_sc_scatter():
    mesh = plsc.VectorSubcoreMesh(core_axis_name="c", subcore_axis_name="s")

    @functools.partial(
        pl.kernel,
        out_type=(
            jax.ShapeDtypeStruct((NC, OWN, D), jnp.float32),
            jax.ShapeDtypeStruct((NS, NB), jnp.float32),
        ),
        mesh=mesh,
        compiler_params=pltpu.CompilerParams(needs_layout_passes=False),
        scratch_types=[
            pltpu.VMEM((CPT, CH), jnp.int32),
            pltpu.VMEM((CPT, CH), jnp.int32),
            pltpu.VMEM((2, CH, D), jnp.float32),
            pltpu.VMEM((2, CH), jnp.int32),
            pltpu.VMEM((NB,), jnp.float32),
            pltpu.VMEM_SHARED((NA, D), jnp.float32),
            pltpu.SemaphoreType.DMA,
            pltpu.SemaphoreType.DMA,
        ],
    )
    def sc_scatter(xl_hbm, sidx_hbm, ridx_hbm, zeros_hbm, zflat_hbm,
                   out_hbm, cnt_hbm,
                   sidx_v, ridx_v, rows_v, rsel_v, hist_v, acc, sem, sem2):
        core = lax.axis_index("c")
        sub = lax.axis_index("s")
        base = core * OWN

        pltpu.sync_copy(zeros_hbm, acc.at[pl.ds(sub * RZT, RZT)])
        pltpu.sync_copy(zflat_hbm, hist_v)
        pltpu.sync_copy(sidx_hbm.at[pl.ds(sub * CPT, CPT)], sidx_v)
        pltpu.sync_copy(ridx_hbm.at[pl.ds(sub * CPT, CPT)], ridx_v)
        plsc.subcore_barrier()

        ones16 = jnp.full((16,), 1.0, jnp.float32)

        pltpu.async_copy(xl_hbm.at[sidx_v.at[0]], rows_v.at[0], sem)

        def chunk(j, carry):
            p = lax.bitwise_and(j, 1)
            pltpu.make_async_copy(
                xl_hbm.at[sidx_v.at[j]], rows_v.at[p], sem).wait()

            for k in range(CH // 16):
                r16 = ridx_v[j, pl.ds(k * 16, 16)]
                rr = r16 - base
                owned = jnp.logical_and(rr >= 0, rr < OWN)
                dump = OWN + lax.bitwise_and(r16, DUMP - 1)
                rsel_v[p, pl.ds(k * 16, 16)] = jnp.where(owned, rr, dump)

            @pl.when(core == 0)
            def _():
                for k in range(CH // 16):
                    r16 = ridx_v[j, pl.ds(k * 16, 16)]
                    plsc.addupdate_scatter(hist_v, [r16], ones16)

            @pl.when(j > 0)
            def _():
                pltpu.make_async_copy(
                    rows_v.at[1 - p], acc.at[rsel_v.at[1 - p]], sem2).wait()

            pltpu.async_copy(rows_v.at[p], acc.at[rsel_v.at[p]], sem2,
                             add=True)

            @pl.when(j < CPT - 1)
            def _():
                pltpu.async_copy(
                    xl_hbm.at[sidx_v.at[j + 1]], rows_v.at[1 - p], sem)

            return carry

        lax.fori_loop(0, CPT, chunk, 0)

        lastp = (CPT - 1) & 1
        pltpu.make_async_copy(
            rows_v.at[lastp], acc.at[rsel_v.at[lastp]], sem2).wait()

        @pl.when(core == 0)
        def _():
            pltpu.sync_copy(hist_v, cnt_hbm.at[sub])

        plsc.subcore_barrier()
        pltpu.sync_copy(acc.at[pl.ds(sub * RWT, RWT)],
                        out_hbm.at[core, pl.ds(sub * RWT, RWT)])

    return sc_scatter


_sc_scatter = _make_sc_scatter()


def kernel(x, adj, key):
    del key
    xl = pl.pallas_call(
        _logmap_body,
        grid=(10,),
        in_specs=[pl.BlockSpec((N // 10, D), lambda i: (i, 0))],
        out_specs=pl.BlockSpec((N // 10, D), lambda i: (i, 0)),
        out_shape=jax.ShapeDtypeStruct((N, D), jnp.float32),
    )(x)

    pad = E_PAD - E
    pi = jnp.arange(pad, dtype=jnp.int32)
    s = jnp.concatenate([adj[0], pi % N])
    r = jnp.concatenate([adj[1], N + (pi % (NB - N))])
    s2 = s.reshape(E_PAD // CH, CH)
    r2 = r.reshape(E_PAD // CH, CH)
    zeros = jnp.zeros((RZT, D), jnp.float32)
    zflat = jnp.zeros((NB,), jnp.float32)

    partial, cnt_planes = _sc_scatter(xl, s2, r2, zeros, zflat)
    sums = partial.reshape(NC * OWN, D)
    counts = cnt_planes.sum(axis=0)[:N, None]

    out = pl.pallas_call(
        _final_body,
        grid=(10,),
        in_specs=[
            pl.BlockSpec((N // 10, D), lambda i: (i, 0)),
            pl.BlockSpec((N // 10, 1), lambda i: (i, 0)),
        ],
        out_specs=pl.BlockSpec((N // 10, D), lambda i: (i, 0)),
        out_shape=jax.ShapeDtypeStruct((N, D), jnp.float32),
    )(sums, counts)
    return out


# speedup vs baseline: 1.4836x; 1.4836x over previous
import functools

import jax
import jax.numpy as jnp
from jax import lax
from jax.experimental import pallas as pl
from jax.experimental.pallas import tpu as pltpu
from jax.experimental.pallas import tpu_sc as plsc

N = 10000
D = 128
E = 320000
MIN_NORM = 1e-15
EPS = 4e-3

NC, NS = 2, 16
CH = 128
CPT = 160
E_PAD = NS * CPT * CH
OWN = 5120
DUMP = 128
NA = OWN + DUMP
RZT = NA // NS
RWT = OWN // NS


def _logmap_body(x_ref, o_ref):
    p = x_ref[...]
    pn = jnp.maximum(jnp.sqrt(jnp.sum(p * p, axis=1, keepdims=True)), MIN_NORM)
    t = jnp.clip(pn, -1.0 + 1e-7, 1.0 - 1e-7)
    o_ref[...] = ((0.5 * jnp.log((1.0 + t) / (1.0 - t))) / pn) * p


def _final_body(p_ref, c_ref, o_ref):
    m = p_ref[...] / jnp.maximum(c_ref[...], 1.0)
    un = jnp.maximum(jnp.sqrt(jnp.sum(m * m, axis=1, keepdims=True)), MIN_NORM)
    e = jnp.tanh(un) * m / un
    en = jnp.maximum(jnp.sqrt(jnp.sum(e * e, axis=1, keepdims=True)), MIN_NORM)
    maxnorm = 1.0 - EPS
    o_ref[...] = jnp.where(en > maxnorm, e / en * maxnorm, e)


def _make_sc_scatter():
    mesh = plsc.VectorSubcoreMesh(core_axis_name="c", subcore_axis_name="s")

    @functools.partial(
        pl.kernel,
        out_type=(
            jax.ShapeDtypeStruct((NC, OWN, D), jnp.float32),
            jax.ShapeDtypeStruct((NC, NS, NA), jnp.float32),
        ),
        mesh=mesh,
        compiler_params=pltpu.CompilerParams(needs_layout_passes=False),
        scratch_types=[
            pltpu.VMEM((CPT, CH), jnp.int32),
            pltpu.VMEM((CPT, CH), jnp.int32),
            pltpu.VMEM((2, CH, D), jnp.float32),
            pltpu.VMEM((NA,), jnp.float32),
            pltpu.VMEM_SHARED((NA, D), jnp.float32),
            pltpu.SemaphoreType.DMA,
            pltpu.SemaphoreType.DMA,
        ],
    )
    def sc_scatter(xl_hbm, sidx_hbm, ridx_hbm, zeros_hbm, zflat_hbm,
                   out_hbm, cnt_hbm,
                   sidx_v, ridx_v, rows_v, hist_v, acc, sem, sem2):
        core = lax.axis_index("c")
        sub = lax.axis_index("s")
        base = core * OWN

        pltpu.sync_copy(zeros_hbm, acc.at[pl.ds(sub * RZT, RZT)])
        pltpu.sync_copy(zflat_hbm, hist_v)
        pltpu.sync_copy(sidx_hbm.at[pl.ds(sub * CPT, CPT)], sidx_v)
        pltpu.sync_copy(ridx_hbm.at[pl.ds(sub * CPT, CPT)], ridx_v)
        plsc.subcore_barrier()

        ones16 = jnp.full((16,), 1.0, jnp.float32)
        iota16 = lax.iota(jnp.int32, 16)

        def compact(j, cur):
            for k in range(CH // 16):
                s16 = sidx_v[j, pl.ds(k * 16, 16)]
                r16 = ridx_v[j, pl.ds(k * 16, 16)]
                rr = r16 - base
                owned = jnp.logical_and(
                    jnp.logical_and(rr >= 0, rr < OWN), r16 < N)
                oi = jnp.where(owned, 1, 0).astype(jnp.int32)
                pos = cur + plsc.cumsum(oi) - oi
                pr = lax.shift_right_logical(pos, 7)
                pc = lax.bitwise_and(pos, CH - 1)
                plsc.store_scatter(sidx_v, [pr, pc], s16, mask=owned)
                plsc.store_scatter(ridx_v, [pr, pc], rr, mask=owned)
                plsc.addupdate_scatter(hist_v, [rr], ones16, mask=owned)
                cur = cur + jnp.sum(oi)
            return cur

        cur = lax.fori_loop(0, CPT, compact, jnp.int32(0))

        for g in range(CH // 16):
            lane = g * 16 + iota16
            pos = cur + lane
            okp = pos < CPT * CH
            pr = lax.shift_right_logical(pos, 7)
            pc = lax.bitwise_and(pos, CH - 1)
            plsc.store_scatter(sidx_v, [pr, pc], lane * 64, mask=okp)
            plsc.store_scatter(ridx_v, [pr, pc], OWN + lane, mask=okp)
        nch = lax.shift_right_logical(cur + CH - 1, 7)

        @pl.when(nch > 0)
        def _():
            pltpu.async_copy(xl_hbm.at[sidx_v.at[0]], rows_v.at[0], sem)

        def chunk(q, carry):
            p = lax.bitwise_and(q, 1)
            pltpu.make_async_copy(
                xl_hbm.at[sidx_v.at[q]], rows_v.at[p], sem).wait()

            @pl.when(q > 0)
            def _():
                pltpu.make_async_copy(
                    rows_v.at[1 - p], acc.at[ridx_v.at[q - 1]], sem2).wait()

            pltpu.async_copy(rows_v.at[p], acc.at[ridx_v.at[q]], sem2,
                             add=True)

            @pl.when(q < nch - 1)
            def _():
                pltpu.async_copy(
                    xl_hbm.at[sidx_v.at[q + 1]], rows_v.at[1 - p], sem)

            return carry

        lax.fori_loop(0, nch, chunk, 0)

        @pl.when(nch > 0)
        def _():
            lastp = lax.bitwise_and(nch - 1, 1)
            pltpu.make_async_copy(
                rows_v.at[lastp], acc.at[ridx_v.at[nch - 1]], sem2).wait()

        pltpu.sync_copy(hist_v, cnt_hbm.at[core, sub])
        plsc.subcore_barrier()
        pltpu.sync_copy(acc.at[pl.ds(sub * RWT, RWT)],
                        out_hbm.at[core, pl.ds(sub * RWT, RWT)])

    return sc_scatter


_sc_scatter = _make_sc_scatter()


def kernel(x, adj, key):
    del key
    xl = pl.pallas_call(
        _logmap_body,
        grid=(10,),
        in_specs=[pl.BlockSpec((N // 10, D), lambda i: (i, 0))],
        out_specs=pl.BlockSpec((N // 10, D), lambda i: (i, 0)),
        out_shape=jax.ShapeDtypeStruct((N, D), jnp.float32),
    )(x)

    pad = E_PAD - E
    pi = jnp.arange(pad, dtype=jnp.int32)
    s = jnp.concatenate([adj[0], pi % N])
    r = jnp.concatenate([adj[1], jnp.full((pad,), N, jnp.int32)])
    s2 = s.reshape(E_PAD // CH, CH)
    r2 = r.reshape(E_PAD // CH, CH)
    zeros = jnp.zeros((RZT, D), jnp.float32)
    zflat = jnp.zeros((NA,), jnp.float32)

    partial, cnt_planes = _sc_scatter(xl, s2, r2, zeros, zflat)
    sums = partial.reshape(NC * OWN, D)
    counts = jnp.concatenate(
        [cnt_planes[0].sum(axis=0)[:OWN], cnt_planes[1].sum(axis=0)[:OWN]]
    )[:N, None]

    out = pl.pallas_call(
        _final_body,
        grid=(10,),
        in_specs=[
            pl.BlockSpec((N // 10, D), lambda i: (i, 0)),
            pl.BlockSpec((N // 10, 1), lambda i: (i, 0)),
        ],
        out_specs=pl.BlockSpec((N // 10, D), lambda i: (i, 0)),
        out_shape=jax.ShapeDtypeStruct((N, D), jnp.float32),
    )(sums, counts)
    return out


# speedup vs baseline: 1.4897x; 1.0042x over previous
import functools

import jax
import jax.numpy as jnp
from jax import lax
from jax.experimental import pallas as pl
from jax.experimental.pallas import tpu as pltpu
from jax.experimental.pallas import tpu_sc as plsc

N = 10000
D = 128
E = 320000
MIN_NORM = 1e-15
EPS = 4e-3

NC, NS = 2, 16
CH = 128
CPT = 160
E_PAD = NS * CPT * CH
OWN = 5120
DUMP = 128
NA = OWN + DUMP
RZT = NA // NS
RWT = OWN // NS


def _logmap_body(x_ref, o_ref):
    p = x_ref[...]
    pn = jnp.maximum(jnp.sqrt(jnp.sum(p * p, axis=1, keepdims=True)), MIN_NORM)
    t = jnp.clip(pn, -1.0 + 1e-7, 1.0 - 1e-7)
    o_ref[...] = ((0.5 * jnp.log((1.0 + t) / (1.0 - t))) / pn) * p


def _final_body(p_ref, c_ref, o_ref):
    m = p_ref[...] / jnp.maximum(c_ref[...], 1.0)
    un = jnp.maximum(jnp.sqrt(jnp.sum(m * m, axis=1, keepdims=True)), MIN_NORM)
    e = jnp.tanh(un) * m / un
    en = jnp.maximum(jnp.sqrt(jnp.sum(e * e, axis=1, keepdims=True)), MIN_NORM)
    maxnorm = 1.0 - EPS
    o_ref[...] = jnp.where(en > maxnorm, e / en * maxnorm, e)


def _make_sc_scatter():
    mesh = plsc.VectorSubcoreMesh(core_axis_name="c", subcore_axis_name="s")

    @functools.partial(
        pl.kernel,
        out_type=(
            jax.ShapeDtypeStruct((NC, OWN, D), jnp.float32),
            jax.ShapeDtypeStruct((NC, NS, NA), jnp.float32),
        ),
        mesh=mesh,
        compiler_params=pltpu.CompilerParams(needs_layout_passes=False),
        scratch_types=[
            pltpu.VMEM((CPT, CH), jnp.int32),
            pltpu.VMEM((CPT, CH), jnp.int32),
            pltpu.VMEM((2, CH, D), jnp.float32),
            pltpu.VMEM((NA,), jnp.float32),
            pltpu.VMEM_SHARED((NA, D), jnp.float32),
            pltpu.SemaphoreType.DMA,
            pltpu.SemaphoreType.DMA,
        ],
    )
    def sc_scatter(xl_hbm, sidx_hbm, ridx_hbm, zeros_hbm, zflat_hbm,
                   out_hbm, cnt_hbm,
                   sidx_v, ridx_v, rows_v, hist_v, acc, sem, sem2):
        core = lax.axis_index("c")
        sub = lax.axis_index("s")
        base = core * OWN

        pltpu.sync_copy(zeros_hbm, acc.at[pl.ds(sub * RZT, RZT)])
        pltpu.sync_copy(zflat_hbm, hist_v)
        pltpu.sync_copy(sidx_hbm.at[pl.ds(sub * CPT, CPT)], sidx_v)
        pltpu.sync_copy(ridx_hbm.at[pl.ds(sub * CPT, CPT)], ridx_v)

        ones16 = jnp.full((16,), 1.0, jnp.float32)
        iota16 = lax.iota(jnp.int32, 16)

        def compact(j, curv):
            for k in range(CH // 16):
                s16 = sidx_v[j, pl.ds(k * 16, 16)]
                r16 = ridx_v[j, pl.ds(k * 16, 16)]
                rr = r16 - base
                owned = jnp.logical_and(
                    jnp.logical_and(rr >= 0, rr < OWN), r16 < N)
                oi = jnp.where(owned, 1, 0).astype(jnp.int32)
                cs = plsc.cumsum(oi)
                pos = curv + cs - oi
                pr = lax.shift_right_logical(pos, 7)
                pc = lax.bitwise_and(pos, CH - 1)
                plsc.store_scatter(sidx_v, [pr, pc], s16, mask=owned)
                plsc.store_scatter(ridx_v, [pr, pc], rr, mask=owned)
                plsc.addupdate_scatter(hist_v, [rr], ones16, mask=owned)
                curv = curv + cs.at[jnp.full((16,), 15, jnp.int32)].get(
                    mode="promise_in_bounds")
            return curv

        curv = lax.fori_loop(0, CPT, compact, jnp.zeros((16,), jnp.int32))

        for g in range(CH // 16):
            lane = g * 16 + iota16
            pos = curv + lane
            okp = pos < CPT * CH
            pr = lax.shift_right_logical(pos, 7)
            pc = lax.bitwise_and(pos, CH - 1)
            plsc.store_scatter(sidx_v, [pr, pc], lane * 64, mask=okp)
            plsc.store_scatter(ridx_v, [pr, pc], OWN + lane, mask=okp)
        cur = jnp.sum(jnp.where(iota16 == 0, curv, 0))
        nch = lax.shift_right_logical(cur + CH - 1, 7)

        plsc.subcore_barrier()

        @pl.when(nch > 0)
        def _():
            pltpu.async_copy(xl_hbm.at[sidx_v.at[0]], rows_v.at[0], sem)

        def chunk(q, carry):
            p = lax.bitwise_and(q, 1)
            pltpu.make_async_copy(
                xl_hbm.at[sidx_v.at[q]], rows_v.at[p], sem).wait()

            @pl.when(q > 0)
            def _():
                pltpu.make_async_copy(
                    rows_v.at[1 - p], acc.at[ridx_v.at[q - 1]], sem2).wait()

            pltpu.async_copy(rows_v.at[p], acc.at[ridx_v.at[q]], sem2,
                             add=True)

            @pl.when(q < nch - 1)
            def _():
                pltpu.async_copy(
                    xl_hbm.at[sidx_v.at[q + 1]], rows_v.at[1 - p], sem)

            return carry

        lax.fori_loop(0, nch, chunk, 0)

        @pl.when(nch > 0)
        def _():
            lastp = lax.bitwise_and(nch - 1, 1)
            pltpu.make_async_copy(
                rows_v.at[lastp], acc.at[ridx_v.at[nch - 1]], sem2).wait()

        pltpu.sync_copy(hist_v, cnt_hbm.at[core, sub])
        plsc.subcore_barrier()
        pltpu.sync_copy(acc.at[pl.ds(sub * RWT, RWT)],
                        out_hbm.at[core, pl.ds(sub * RWT, RWT)])

    return sc_scatter


_sc_scatter = _make_sc_scatter()


def kernel(x, adj, key):
    del key
    xl = pl.pallas_call(
        _logmap_body,
        grid=(10,),
        in_specs=[pl.BlockSpec((N // 10, D), lambda i: (i, 0))],
        out_specs=pl.BlockSpec((N // 10, D), lambda i: (i, 0)),
        out_shape=jax.ShapeDtypeStruct((N, D), jnp.float32),
    )(x)

    pad = E_PAD - E
    pi = jnp.arange(pad, dtype=jnp.int32)
    s = jnp.concatenate([adj[0], pi % N])
    r = jnp.concatenate([adj[1], jnp.full((pad,), N, jnp.int32)])
    s2 = s.reshape(E_PAD // CH, CH)
    r2 = r.reshape(E_PAD // CH, CH)
    zeros = jnp.zeros((RZT, D), jnp.float32)
    zflat = jnp.zeros((NA,), jnp.float32)

    partial, cnt_planes = _sc_scatter(xl, s2, r2, zeros, zflat)
    sums = partial.reshape(NC * OWN, D)
    counts = jnp.concatenate(
        [cnt_planes[0].sum(axis=0)[:OWN], cnt_planes[1].sum(axis=0)[:OWN]]
    )[:N, None]

    out = pl.pallas_call(
        _final_body,
        grid=(10,),
        in_specs=[
            pl.BlockSpec((N // 10, D), lambda i: (i, 0)),
            pl.BlockSpec((N // 10, 1), lambda i: (i, 0)),
        ],
        out_specs=pl.BlockSpec((N // 10, D), lambda i: (i, 0)),
        out_shape=jax.ShapeDtypeStruct((N, D), jnp.float32),
    )(sums, counts)
    return out


# speedup vs baseline: 1.5157x; 1.0175x over previous
import functools

import jax
import jax.numpy as jnp
from jax import lax
from jax.experimental import pallas as pl
from jax.experimental.pallas import tpu as pltpu
from jax.experimental.pallas import tpu_sc as plsc

N = 10000
D = 128
E = 320000
MIN_NORM = 1e-15
EPS = 4e-3

NC, NS = 2, 16
CH = 128
CPT = 160
E_PAD = NS * CPT * CH
OWN = 5120
DUMP = 128
NA = OWN + DUMP
RZT = NA // NS
RWT = OWN // NS


def _logmap_body(x_ref, o_ref):
    p = x_ref[...]
    pn = jnp.maximum(jnp.sqrt(jnp.sum(p * p, axis=1, keepdims=True)), MIN_NORM)
    t = jnp.clip(pn, -1.0 + 1e-7, 1.0 - 1e-7)
    o_ref[...] = ((0.5 * jnp.log((1.0 + t) / (1.0 - t))) / pn) * p


def _final_body(p_ref, c_ref, o_ref):
    m = p_ref[...] / jnp.maximum(c_ref[...], 1.0)
    un = jnp.maximum(jnp.sqrt(jnp.sum(m * m, axis=1, keepdims=True)), MIN_NORM)
    e = jnp.tanh(un) * m / un
    en = jnp.maximum(jnp.sqrt(jnp.sum(e * e, axis=1, keepdims=True)), MIN_NORM)
    maxnorm = 1.0 - EPS
    o_ref[...] = jnp.where(en > maxnorm, e / en * maxnorm, e)


def _make_sc_scatter():
    mesh = plsc.VectorSubcoreMesh(core_axis_name="c", subcore_axis_name="s")

    @functools.partial(
        pl.kernel,
        out_type=(
            jax.ShapeDtypeStruct((NC, OWN, D), jnp.float32),
            jax.ShapeDtypeStruct((NC, NS, NA), jnp.float32),
        ),
        mesh=mesh,
        compiler_params=pltpu.CompilerParams(needs_layout_passes=False),
        scratch_types=[
            pltpu.VMEM((CPT, CH), jnp.int32),
            pltpu.VMEM((CPT, CH), jnp.int32),
            pltpu.VMEM((2, CH, D), jnp.float32),
            pltpu.VMEM((NA,), jnp.float32),
            pltpu.VMEM_SHARED((NA, D), jnp.float32),
            pltpu.SemaphoreType.DMA,
            pltpu.SemaphoreType.DMA,
            pltpu.SemaphoreType.DMA,
        ],
    )
    def sc_scatter(xl_hbm, sidx_hbm, ridx_hbm, zeros_hbm, zflat_hbm,
                   out_hbm, cnt_hbm,
                   sidx_v, ridx_v, rows_v, hist_v, acc, sem, sem2, sem3):
        core = lax.axis_index("c")
        sub = lax.axis_index("s")
        base = core * OWN

        pltpu.sync_copy(zeros_hbm, acc.at[pl.ds(sub * RZT, RZT)])
        pltpu.sync_copy(zflat_hbm, hist_v)
        pltpu.sync_copy(sidx_hbm.at[pl.ds(sub * CPT, CPT)], sidx_v)
        pltpu.sync_copy(ridx_hbm.at[pl.ds(sub * CPT, CPT)], ridx_v)

        ones16 = jnp.full((16,), 1.0, jnp.float32)
        iota16 = lax.iota(jnp.int32, 16)

        def compact(j, curv):
            for k in range(CH // 16):
                s16 = sidx_v[j, pl.ds(k * 16, 16)]
                r16 = ridx_v[j, pl.ds(k * 16, 16)]
                rr = r16 - base
                owned = jnp.logical_and(
                    jnp.logical_and(rr >= 0, rr < OWN), r16 < N)
                oi = jnp.where(owned, 1, 0).astype(jnp.int32)
                cs = plsc.cumsum(oi)
                pos = curv + cs - oi
                pr = lax.shift_right_logical(pos, 7)
                pc = lax.bitwise_and(pos, CH - 1)
                plsc.store_scatter(sidx_v, [pr, pc], s16, mask=owned)
                plsc.store_scatter(ridx_v, [pr, pc], rr, mask=owned)
                plsc.addupdate_scatter(hist_v, [rr], ones16, mask=owned)
                curv = curv + cs.at[jnp.full((16,), 15, jnp.int32)].get(
                    mode="promise_in_bounds")
            return curv

        curv = lax.fori_loop(0, CPT, compact, jnp.zeros((16,), jnp.int32))

        for g in range(CH // 16):
            lane = g * 16 + iota16
            pos = curv + lane
            okp = pos < CPT * CH
            pr = lax.shift_right_logical(pos, 7)
            pc = lax.bitwise_and(pos, CH - 1)
            plsc.store_scatter(sidx_v, [pr, pc], lane * 64, mask=okp)
            plsc.store_scatter(ridx_v, [pr, pc], OWN + lane, mask=okp)
        cur = jnp.sum(jnp.where(iota16 == 0, curv, 0))
        nch = lax.shift_right_logical(cur + CH - 1, 7)

        plsc.subcore_barrier()

        HF = CH // 2

        def gather2(q, p):
            pltpu.async_copy(xl_hbm.at[sidx_v.at[q, pl.ds(0, HF)]],
                             rows_v.at[p, pl.ds(0, HF)], sem)
            pltpu.async_copy(xl_hbm.at[sidx_v.at[q, pl.ds(HF, HF)]],
                             rows_v.at[p, pl.ds(HF, HF)], sem3)

        def gather2_wait(q, p):
            pltpu.make_async_copy(xl_hbm.at[sidx_v.at[q, pl.ds(0, HF)]],
                                  rows_v.at[p, pl.ds(0, HF)], sem).wait()
            pltpu.make_async_copy(xl_hbm.at[sidx_v.at[q, pl.ds(HF, HF)]],
                                  rows_v.at[p, pl.ds(HF, HF)], sem3).wait()

        @pl.when(nch > 0)
        def _():
            gather2(0, 0)

        def chunk(q, carry):
            p = lax.bitwise_and(q, 1)
            gather2_wait(q, p)

            @pl.when(q > 0)
            def _():
                pltpu.make_async_copy(
                    rows_v.at[1 - p], acc.at[ridx_v.at[q - 1]], sem2).wait()

            pltpu.async_copy(rows_v.at[p], acc.at[ridx_v.at[q]], sem2,
                             add=True)

            @pl.when(q < nch - 1)
            def _():
                gather2(q + 1, 1 - p)

            return carry

        lax.fori_loop(0, nch, chunk, 0)

        @pl.when(nch > 0)
        def _():
            lastp = lax.bitwise_and(nch - 1, 1)
            pltpu.make_async_copy(
                rows_v.at[lastp], acc.at[ridx_v.at[nch - 1]], sem2).wait()

        pltpu.sync_copy(hist_v, cnt_hbm.at[core, sub])
        plsc.subcore_barrier()
        pltpu.sync_copy(acc.at[pl.ds(sub * RWT, RWT)],
                        out_hbm.at[core, pl.ds(sub * RWT, RWT)])

    return sc_scatter


_sc_scatter = _make_sc_scatter()


def kernel(x, adj, key):
    del key
    xl = pl.pallas_call(
        _logmap_body,
        grid=(10,),
        in_specs=[pl.BlockSpec((N // 10, D), lambda i: (i, 0))],
        out_specs=pl.BlockSpec((N // 10, D), lambda i: (i, 0)),
        out_shape=jax.ShapeDtypeStruct((N, D), jnp.float32),
    )(x)

    pad = E_PAD - E
    pi = jnp.arange(pad, dtype=jnp.int32)
    s = jnp.concatenate([adj[0], pi % N])
    r = jnp.concatenate([adj[1], jnp.full((pad,), N, jnp.int32)])
    s2 = s.reshape(E_PAD // CH, CH)
    r2 = r.reshape(E_PAD // CH, CH)
    zeros = jnp.zeros((RZT, D), jnp.float32)
    zflat = jnp.zeros((NA,), jnp.float32)

    partial, cnt_planes = _sc_scatter(xl, s2, r2, zeros, zflat)
    sums = partial.reshape(NC * OWN, D)
    counts = jnp.concatenate(
        [cnt_planes[0].sum(axis=0)[:OWN], cnt_planes[1].sum(axis=0)[:OWN]]
    )[:N, None]

    out = pl.pallas_call(
        _final_body,
        grid=(10,),
        in_specs=[
            pl.BlockSpec((N // 10, D), lambda i: (i, 0)),
            pl.BlockSpec((N // 10, 1), lambda i: (i, 0)),
        ],
        out_specs=pl.BlockSpec((N // 10, D), lambda i: (i, 0)),
        out_shape=jax.ShapeDtypeStruct((N, D), jnp.float32),
    )(sums, counts)
    return out
